# baseline (speedup 1.0000x reference)
import functools

import jax
import jax.numpy as jnp
from jax import lax
from jax.experimental import pallas as pl
from jax.experimental.pallas import tpu as pltpu
from jax.experimental.pallas import tpu_sc as plsc

CAP = 64


def _sc_gather(table, idx2d, d):
    nch = idx2d.shape[0]
    info = plsc.get_sparse_core_info()
    nw = info.num_cores * info.num_subcores
    per_w = nch // nw
    mesh = plsc.VectorSubcoreMesh(core_axis_name="c", subcore_axis_name="s")

    n_grp = per_w // 4

    @functools.partial(
        pl.kernel,
        out_type=jax.ShapeDtypeStruct((nch * 128, d), jnp.float32),
        mesh=mesh,
        scratch_types=[
            pltpu.VMEM((4, 128), jnp.int32),
            pltpu.VMEM((512, d), jnp.float32),
            pltpu.SemaphoreType.DMA,
        ],
    )
    def gather_k(table_hbm, idx_hbm, out_hbm, idx_v, rows_v, sem):
        wid = lax.axis_index("s") * info.num_cores + lax.axis_index("c")
        base = wid * per_w

        def body(g, carry):
            gbase = base + g * 4
            pltpu.sync_copy(idx_hbm.at[pl.ds(gbase, 4)], idx_v)
            handles = [
                pltpu.async_copy(
                    table_hbm.at[idx_v.at[j]],
                    rows_v.at[pl.ds(j * 128, 128)],
                    sem,
                )
                for j in range(4)
            ]
            for hnd in handles:
                hnd.wait()
            pltpu.sync_copy(rows_v, out_hbm.at[pl.ds(gbase * 128, 512)])
            return carry

        lax.fori_loop(0, n_grp, body, 0)

    return gather_k(table, idx2d)


def _mm_body(x_ref, w_ref, b_ref, o_ref):
    o_ref[...] = (
        jnp.dot(x_ref[...], w_ref[...], preferred_element_type=jnp.float32)
        + b_ref[...]
    )


def _matmul_bias(x, w, b, row_block):
    n, f = x.shape
    k = w.shape[1]
    grid = (n // row_block,)
    return pl.pallas_call(
        _mm_body,
        grid=grid,
        in_specs=[
            pl.BlockSpec((row_block, f), lambda i: (i, 0)),
            pl.BlockSpec((f, k), lambda i: (0, 0)),
            pl.BlockSpec((1, k), lambda i: (0, 0)),
        ],
        out_specs=pl.BlockSpec((row_block, k), lambda i: (i, 0)),
        out_shape=jax.ShapeDtypeStruct((n, k), jnp.float32),
    )(x, w, b.reshape(1, k))


def _bitonic_median(x, cnt):
    j = lax.broadcasted_iota(jnp.int32, x.shape, 1)
    cnt3 = cnt[:, None, :]
    x = jnp.where(j < cnt3, x, jnp.inf)

    def roll1(v, s):
        return jnp.concatenate([v[:, -s:, :], v[:, :-s, :]], axis=1)

    n = x.shape[1]
    k = 2
    while k <= n:
        s = k // 2
        while s >= 1:
            up = roll1(x, -s)
            dn = roll1(x, s)
            low_half = (j & s) == 0
            partner = jnp.where(low_half, up, dn)
            asc = (j & k) == 0
            keep_min = asc == low_half
            x = jnp.where(
                keep_min, jnp.minimum(x, partner), jnp.maximum(x, partner)
            )
            s //= 2
        k *= 2

    kk = (cnt3 - 1) >> 1
    return jnp.sum(jnp.where(j == kk, x, 0.0), axis=1)


def _med1_body(p_ref, c_ref, o_ref):
    med = _bitonic_median(p_ref[...], c_ref[...])
    o_ref[...] = jnp.where(med > 0, med, jnp.exp(med) - 1.0)


def _med2_body(p_ref, c_ref, o_ref):
    o_ref[...] = _bitonic_median(p_ref[...], c_ref[...])


def _median_call(body, padded, cntb, nb):
    n, cap, l = padded.shape
    grid = (n // nb,)
    return pl.pallas_call(
        body,
        grid=grid,
        in_specs=[
            pl.BlockSpec((nb, cap, l), lambda i: (i, 0, 0)),
            pl.BlockSpec((nb, l), lambda i: (i, 0)),
        ],
        out_specs=pl.BlockSpec((nb, l), lambda i: (i, 0)),
        out_shape=jax.ShapeDtypeStruct((n, l), jnp.float32),
    )(padded, cntb)


def _softmax_body(x_ref, o_ref):
    x = x_ref[...]
    m = jnp.max(x, axis=-1, keepdims=True)
    e = jnp.exp(x - m)
    o_ref[...] = e / jnp.sum(e, axis=-1, keepdims=True)


def _softmax16(x):
    r = x.shape[0]
    nb = 25
    return pl.pallas_call(
        _softmax_body,
        grid=(r // nb,),
        in_specs=[pl.BlockSpec((nb, 16, 16), lambda i: (i, 0, 0))],
        out_specs=pl.BlockSpec((nb, 16, 16), lambda i: (i, 0, 0)),
        out_shape=jax.ShapeDtypeStruct(x.shape, jnp.float32),
    )(x)


def _segmed_exact(msgs, dstv, n):
    counts = jnp.bincount(dstv, length=n)
    starts = jnp.cumsum(counts) - counts
    has = counts > 0
    med_pos = jnp.where(has, starts + (counts - 1) // 2, 0)

    def col(v):
        o = jnp.lexsort((v, dstv))
        return v[o][med_pos]

    med = jax.vmap(col, in_axes=1, out_axes=1)(msgs)
    return jnp.where(has[:, None], med, 0.0)


def kernel(X, ei_feat, batch, W1, b1, W2, b2):
    n, f = X.shape
    kdim = W2.shape[1]
    e = ei_feat.shape[1]
    src = ei_feat[0]
    dst = ei_feat[1]

    counts = jnp.zeros((n,), jnp.int32).at[dst].add(1)
    starts = jnp.cumsum(counts) - counts
    order = jnp.argsort(dst)
    sdst = dst[order]
    slot = jnp.arange(e, dtype=jnp.int32) - starts[sdst]
    ssrc = src[order]
    valid = slot < CAP
    overflow = jnp.logical_not(jnp.all(valid))

    n1p = ((n * CAP + 16383) // 16384) * 16384 // CAP
    pos1 = jnp.where(valid, sdst * CAP + slot, n1p * CAP)
    gidx1 = jnp.zeros((n1p * CAP,), jnp.int32).at[pos1].set(ssrc, mode="drop")
    rows2 = ((n // 8 + 127) // 128) * 128
    pos2 = jnp.where(
        valid, (sdst >> 3) * (CAP * 8) + slot * 8 + (sdst & 7), rows2 * CAP * 8
    )
    gidx2 = jnp.zeros((rows2 * CAP * 8,), jnp.int32).at[pos2].set(
        ssrc, mode="drop"
    )

    cnt1 = jnp.broadcast_to(
        jnp.zeros((n1p,), jnp.int32).at[:n].set(counts)[:, None], (n1p, f)
    )
    cpad = jnp.zeros((rows2 * 8,), jnp.int32).at[: n].set(counts)
    cnt2 = jnp.broadcast_to(
        cpad.reshape(rows2, 8)[:, :, None], (rows2, 8, kdim)
    ).reshape(rows2, 8 * kdim)

    h = _matmul_bias(X, W1, b1, row_block=1000)

    def fast(_):
        p1 = _sc_gather(h, gidx1.reshape(-1, 128), f).reshape(n1p, CAP, f)
        hh = _median_call(_med1_body, p1, cnt1, nb=16)
        z = _matmul_bias(hh[:n], W2, b2, row_block=1000)
        p2 = z[gidx2].reshape(rows2, CAP, 8 * kdim)
        m2 = _median_call(_med2_body, p2, cnt2, nb=16)
        m2 = m2[: n // 8].reshape(n // 16, 16, kdim)
        return _softmax16(m2).reshape(n, kdim)

    def slow(_):
        hm = jax.nn.elu(_segmed_exact(h[src], dst, n))
        z = hm @ W2 + b2
        return jax.nn.softmax(_segmed_exact(z[src], dst, n), axis=1)

    return lax.cond(overflow, slow, fast, None)


# speedup vs baseline: 2.3682x; 2.3682x over previous
import functools

import jax
import jax.numpy as jnp
from jax import lax
from jax.experimental import pallas as pl
from jax.experimental.pallas import tpu as pltpu
from jax.experimental.pallas import tpu_sc as plsc

CAP = 64


def _sc_gather(table, idx2d, d):
    nch = idx2d.shape[0]
    info = plsc.get_sparse_core_info()
    nw = info.num_cores * info.num_subcores
    per_w = nch // nw
    mesh = plsc.VectorSubcoreMesh(core_axis_name="c", subcore_axis_name="s")

    n_grp = per_w // 2
    t_rows = table.shape[0]
    per_sub = (t_rows // info.num_subcores) & ~7
    tail_off = per_sub * info.num_subcores
    tail = t_rows - tail_off

    @functools.partial(
        pl.kernel,
        out_type=jax.ShapeDtypeStruct((nch * 128, d), jnp.float32),
        mesh=mesh,
        scratch_types=[
            pltpu.VMEM((2, 128), jnp.int32),
            pltpu.VMEM((256, d), jnp.float32),
            pltpu.VMEM_SHARED((t_rows, d), jnp.float32),
            pltpu.SemaphoreType.DMA,
        ],
    )
    def gather_k(table_hbm, idx_hbm, out_hbm, idx_v, rows_v, tab_s, sem):
        sid = lax.axis_index("s")
        wid = sid * info.num_cores + lax.axis_index("c")
        base = wid * per_w
        pltpu.sync_copy(
            table_hbm.at[pl.ds(sid * per_sub, per_sub)],
            tab_s.at[pl.ds(sid * per_sub, per_sub)],
        )
        if tail:
            @pl.when(sid == 0)
            def _stage_tail():
                pltpu.sync_copy(
                    table_hbm.at[pl.ds(tail_off, tail)],
                    tab_s.at[pl.ds(tail_off, tail)],
                )
        plsc.subcore_barrier()

        def body(g, carry):
            gbase = base + g * 2
            pltpu.sync_copy(idx_hbm.at[pl.ds(gbase, 2)], idx_v)
            handles = [
                pltpu.async_copy(
                    tab_s.at[idx_v.at[j]],
                    rows_v.at[pl.ds(j * 128, 128)],
                    sem,
                )
                for j in range(2)
            ]
            for hnd in handles:
                hnd.wait()
            pltpu.sync_copy(rows_v, out_hbm.at[pl.ds(gbase * 128, 256)])
            return carry

        lax.fori_loop(0, n_grp, body, 0)

    return gather_k(table, idx2d)


def _mm_body(x_ref, w_ref, b_ref, o_ref):
    o_ref[...] = (
        jnp.dot(x_ref[...], w_ref[...], preferred_element_type=jnp.float32)
        + b_ref[...]
    )


def _matmul_bias(x, w, b, row_block):
    n, f = x.shape
    k = w.shape[1]
    grid = (n // row_block,)
    return pl.pallas_call(
        _mm_body,
        grid=grid,
        in_specs=[
            pl.BlockSpec((row_block, f), lambda i: (i, 0)),
            pl.BlockSpec((f, k), lambda i: (0, 0)),
            pl.BlockSpec((1, k), lambda i: (0, 0)),
        ],
        out_specs=pl.BlockSpec((row_block, k), lambda i: (i, 0)),
        out_shape=jax.ShapeDtypeStruct((n, k), jnp.float32),
    )(x, w, b.reshape(1, k))


def _bitonic_median(x, cnt):
    j = lax.broadcasted_iota(jnp.int32, x.shape, 1)
    cnt3 = cnt[:, None, :]
    x = jnp.where(j < cnt3, x, jnp.inf)

    def roll1(v, s):
        return jnp.concatenate([v[:, -s:, :], v[:, :-s, :]], axis=1)

    n = x.shape[1]
    k = 2
    while k <= n:
        s = k // 2
        while s >= 1:
            up = roll1(x, -s)
            dn = roll1(x, s)
            low_half = (j & s) == 0
            partner = jnp.where(low_half, up, dn)
            asc = (j & k) == 0
            keep_min = asc == low_half
            x = jnp.where(
                keep_min, jnp.minimum(x, partner), jnp.maximum(x, partner)
            )
            s //= 2
        k *= 2

    kk = (cnt3 - 1) >> 1
    return jnp.sum(jnp.where(j == kk, x, 0.0), axis=1)


def _med1_body(p_ref, c_ref, o_ref):
    med = _bitonic_median(p_ref[...], c_ref[...])
    o_ref[...] = jnp.where(med > 0, med, jnp.exp(med) - 1.0)


def _med2_body(p_ref, c_ref, o_ref):
    o_ref[...] = _bitonic_median(p_ref[...], c_ref[...])


def _median_call(body, padded, cntb, nb):
    n, cap, l = padded.shape
    grid = (n // nb,)
    return pl.pallas_call(
        body,
        grid=grid,
        in_specs=[
            pl.BlockSpec((nb, cap, l), lambda i: (i, 0, 0)),
            pl.BlockSpec((nb, l), lambda i: (i, 0)),
        ],
        out_specs=pl.BlockSpec((nb, l), lambda i: (i, 0)),
        out_shape=jax.ShapeDtypeStruct((n, l), jnp.float32),
    )(padded, cntb)


def _softmax_body(x_ref, o_ref):
    x = x_ref[...]
    m = jnp.max(x, axis=-1, keepdims=True)
    e = jnp.exp(x - m)
    o_ref[...] = e / jnp.sum(e, axis=-1, keepdims=True)


def _softmax16(x):
    r = x.shape[0]
    nb = 25
    return pl.pallas_call(
        _softmax_body,
        grid=(r // nb,),
        in_specs=[pl.BlockSpec((nb, 16, 16), lambda i: (i, 0, 0))],
        out_specs=pl.BlockSpec((nb, 16, 16), lambda i: (i, 0, 0)),
        out_shape=jax.ShapeDtypeStruct(x.shape, jnp.float32),
    )(x)


def _segmed_exact(msgs, dstv, n):
    counts = jnp.bincount(dstv, length=n)
    starts = jnp.cumsum(counts) - counts
    has = counts > 0
    med_pos = jnp.where(has, starts + (counts - 1) // 2, 0)

    def col(v):
        o = jnp.lexsort((v, dstv))
        return v[o][med_pos]

    med = jax.vmap(col, in_axes=1, out_axes=1)(msgs)
    return jnp.where(has[:, None], med, 0.0)


def kernel(X, ei_feat, batch, W1, b1, W2, b2):
    n, f = X.shape
    kdim = W2.shape[1]
    e = ei_feat.shape[1]
    src = ei_feat[0]
    dst = ei_feat[1]

    counts = jnp.zeros((n,), jnp.int32).at[dst].add(1)
    starts = jnp.cumsum(counts) - counts
    order = jnp.argsort(dst)
    sdst = dst[order]
    slot = jnp.arange(e, dtype=jnp.int32) - starts[sdst]
    ssrc = src[order]
    valid = slot < CAP
    overflow = jnp.logical_not(jnp.all(valid))

    n1p = ((n * CAP + 16383) // 16384) * 16384 // CAP
    pos1 = jnp.where(valid, sdst * CAP + slot, n1p * CAP)
    gidx1 = jnp.zeros((n1p * CAP,), jnp.int32).at[pos1].set(ssrc, mode="drop")
    rows2 = ((n // 8 + 127) // 128) * 128
    pos2 = jnp.where(
        valid, (sdst >> 3) * (CAP * 8) + slot * 8 + (sdst & 7), rows2 * CAP * 8
    )
    gidx2 = jnp.zeros((rows2 * CAP * 8,), jnp.int32).at[pos2].set(
        ssrc, mode="drop"
    )

    cnt1 = jnp.broadcast_to(
        jnp.zeros((n1p,), jnp.int32).at[:n].set(counts)[:, None], (n1p, f)
    )
    cpad = jnp.zeros((rows2 * 8,), jnp.int32).at[: n].set(counts)
    cnt2 = jnp.broadcast_to(
        cpad.reshape(rows2, 8)[:, :, None], (rows2, 8, kdim)
    ).reshape(rows2, 8 * kdim)

    h = _matmul_bias(X, W1, b1, row_block=1000)

    def fast(_):
        p1 = _sc_gather(h, gidx1.reshape(-1, 128), f).reshape(n1p, CAP, f)
        hh = _median_call(_med1_body, p1, cnt1, nb=16)
        z = _matmul_bias(hh[:n], W2, b2, row_block=1000)
        p2 = z[gidx2].reshape(rows2, CAP, 8 * kdim)
        m2 = _median_call(_med2_body, p2, cnt2, nb=16)
        m2 = m2[: n // 8].reshape(n // 16, 16, kdim)
        return _softmax16(m2).reshape(n, kdim)

    def slow(_):
        hm = jax.nn.elu(_segmed_exact(h[src], dst, n))
        z = hm @ W2 + b2
        return jax.nn.softmax(_segmed_exact(z[src], dst, n), axis=1)

    return lax.cond(overflow, slow, fast, None)


# speedup vs baseline: 2.3771x; 1.0037x over previous
import functools

import jax
import jax.numpy as jnp
from jax import lax
from jax.experimental import pallas as pl
from jax.experimental.pallas import tpu as pltpu
from jax.experimental.pallas import tpu_sc as plsc

CAP = 64


def _sc_gather(table, idx2d, d):
    nch = idx2d.shape[0]
    info = plsc.get_sparse_core_info()
    nw = info.num_cores * info.num_subcores
    per_w = nch // nw
    mesh = plsc.VectorSubcoreMesh(core_axis_name="c", subcore_axis_name="s")

    n_grp = per_w // 2
    t_rows = table.shape[0]
    per_sub = (t_rows // info.num_subcores) & ~7
    tail_off = per_sub * info.num_subcores
    tail = t_rows - tail_off

    @functools.partial(
        pl.kernel,
        out_type=jax.ShapeDtypeStruct((nch * 128, d), jnp.float32),
        mesh=mesh,
        scratch_types=[
            pltpu.VMEM((2, 128), jnp.int32),
            pltpu.VMEM((256, d), jnp.float32),
            pltpu.VMEM_SHARED((t_rows, d), jnp.float32),
            pltpu.SemaphoreType.DMA,
        ],
    )
    def gather_k(table_hbm, idx_hbm, out_hbm, idx_v, rows_v, tab_s, sem):
        sid = lax.axis_index("s")
        wid = sid * info.num_cores + lax.axis_index("c")
        base = wid * per_w
        pltpu.sync_copy(
            table_hbm.at[pl.ds(sid * per_sub, per_sub)],
            tab_s.at[pl.ds(sid * per_sub, per_sub)],
        )
        if tail:
            @pl.when(sid == 0)
            def _stage_tail():
                pltpu.sync_copy(
                    table_hbm.at[pl.ds(tail_off, tail)],
                    tab_s.at[pl.ds(tail_off, tail)],
                )
        plsc.subcore_barrier()

        def body(g, carry):
            gbase = base + g * 2
            pltpu.sync_copy(idx_hbm.at[pl.ds(gbase, 2)], idx_v)
            handles = [
                pltpu.async_copy(
                    tab_s.at[idx_v.at[j]],
                    rows_v.at[pl.ds(j * 128, 128)],
                    sem,
                )
                for j in range(2)
            ]
            for hnd in handles:
                hnd.wait()
            pltpu.sync_copy(rows_v, out_hbm.at[pl.ds(gbase * 128, 256)])
            return carry

        lax.fori_loop(0, n_grp, body, 0)

    return gather_k(table, idx2d)


def _mm_body(x_ref, w_ref, b_ref, o_ref):
    o_ref[...] = (
        jnp.dot(x_ref[...], w_ref[...], preferred_element_type=jnp.float32)
        + b_ref[...]
    )


def _matmul_bias(x, w, b, row_block):
    n, f = x.shape
    k = w.shape[1]
    grid = (n // row_block,)
    return pl.pallas_call(
        _mm_body,
        grid=grid,
        in_specs=[
            pl.BlockSpec((row_block, f), lambda i: (i, 0)),
            pl.BlockSpec((f, k), lambda i: (0, 0)),
            pl.BlockSpec((1, k), lambda i: (0, 0)),
        ],
        out_specs=pl.BlockSpec((row_block, k), lambda i: (i, 0)),
        out_shape=jax.ShapeDtypeStruct((n, k), jnp.float32),
    )(x, w, b.reshape(1, k))


def _bitonic_median(x, cnt):
    j = lax.broadcasted_iota(jnp.int32, x.shape, 1)
    cnt3 = cnt[:, None, :]
    x = jnp.where(j < cnt3, x, jnp.inf)

    def roll1(v, s):
        return jnp.concatenate([v[:, -s:, :], v[:, :-s, :]], axis=1)

    n = x.shape[1]
    k = 2
    while k <= n:
        s = k // 2
        while s >= 1:
            up = roll1(x, -s)
            dn = roll1(x, s)
            low_half = (j & s) == 0
            partner = jnp.where(low_half, up, dn)
            asc = (j & k) == 0
            keep_min = asc == low_half
            x = jnp.where(
                keep_min, jnp.minimum(x, partner), jnp.maximum(x, partner)
            )
            s //= 2
        k *= 2

    kk = (cnt3 - 1) >> 1
    return jnp.sum(jnp.where(j == kk, x, 0.0), axis=1)


def _med1_body(p_ref, c_ref, o_ref):
    med = _bitonic_median(p_ref[...], c_ref[...])
    o_ref[...] = jnp.where(med > 0, med, jnp.exp(med) - 1.0)


def _med2_body(p_ref, c_ref, o_ref):
    o_ref[...] = _bitonic_median(p_ref[...], c_ref[...])


def _median_call(body, padded, cntb, nb):
    n, cap, l = padded.shape
    grid = (n // nb,)
    return pl.pallas_call(
        body,
        grid=grid,
        in_specs=[
            pl.BlockSpec((nb, cap, l), lambda i: (i, 0, 0)),
            pl.BlockSpec((nb, l), lambda i: (i, 0)),
        ],
        out_specs=pl.BlockSpec((nb, l), lambda i: (i, 0)),
        out_shape=jax.ShapeDtypeStruct((n, l), jnp.float32),
    )(padded, cntb)


def _softmax_body(x_ref, o_ref):
    x = x_ref[...]
    m = jnp.max(x, axis=-1, keepdims=True)
    e = jnp.exp(x - m)
    o_ref[...] = e / jnp.sum(e, axis=-1, keepdims=True)


def _softmax16(x):
    r = x.shape[0]
    nb = 25
    return pl.pallas_call(
        _softmax_body,
        grid=(r // nb,),
        in_specs=[pl.BlockSpec((nb, 16, 16), lambda i: (i, 0, 0))],
        out_specs=pl.BlockSpec((nb, 16, 16), lambda i: (i, 0, 0)),
        out_shape=jax.ShapeDtypeStruct(x.shape, jnp.float32),
    )(x)


def _segmed_exact(msgs, dstv, n):
    counts = jnp.bincount(dstv, length=n)
    starts = jnp.cumsum(counts) - counts
    has = counts > 0
    med_pos = jnp.where(has, starts + (counts - 1) // 2, 0)

    def col(v):
        o = jnp.lexsort((v, dstv))
        return v[o][med_pos]

    med = jax.vmap(col, in_axes=1, out_axes=1)(msgs)
    return jnp.where(has[:, None], med, 0.0)


def kernel(X, ei_feat, batch, W1, b1, W2, b2):
    n, f = X.shape
    kdim = W2.shape[1]
    e = ei_feat.shape[1]
    src = ei_feat[0]
    dst = ei_feat[1]

    counts = jnp.zeros((n,), jnp.int32).at[dst].add(1)
    starts = jnp.cumsum(counts) - counts
    order = jnp.argsort(dst)
    sdst = dst[order]
    slot = jnp.arange(e, dtype=jnp.int32) - starts[sdst]
    ssrc = src[order]
    valid = slot < CAP
    overflow = jnp.logical_not(jnp.all(valid))

    n1p = ((n * CAP + 16383) // 16384) * 16384 // CAP
    pos1 = jnp.where(valid, sdst * CAP + slot, n1p * CAP)
    gidx1 = jnp.zeros((n1p * CAP,), jnp.int32).at[pos1].set(ssrc, mode="drop")
    rows2 = ((n // 8 + 127) // 128) * 128
    pos2 = jnp.where(
        valid, (sdst >> 3) * (CAP * 8) + slot * 8 + (sdst & 7), rows2 * CAP * 8
    )
    gidx2 = jnp.zeros((rows2 * CAP * 8,), jnp.int32).at[pos2].set(
        ssrc, mode="drop"
    )

    cnt1 = jnp.broadcast_to(
        jnp.zeros((n1p,), jnp.int32).at[:n].set(counts)[:, None], (n1p, f)
    )
    cpad = jnp.zeros((rows2 * 8,), jnp.int32).at[: n].set(counts)
    cnt2 = jnp.broadcast_to(
        cpad.reshape(rows2, 8)[:, :, None], (rows2, 8, kdim)
    ).reshape(rows2, 8 * kdim)

    h = _matmul_bias(X, W1, b1, row_block=1000)

    def fast(_):
        p1 = _sc_gather(h, gidx1.reshape(-1, 128), f).reshape(n1p, CAP, f)
        hh = _median_call(_med1_body, p1, cnt1, nb=32)
        z = _matmul_bias(hh[:n], W2, b2, row_block=1000)
        p2 = z[gidx2].reshape(rows2, CAP, 8 * kdim)
        m2 = _median_call(_med2_body, p2, cnt2, nb=32)
        m2 = m2[: n // 8].reshape(n // 16, 16, kdim)
        return _softmax16(m2).reshape(n, kdim)

    def slow(_):
        hm = jax.nn.elu(_segmed_exact(h[src], dst, n))
        z = hm @ W2 + b2
        return jax.nn.softmax(_segmed_exact(z[src], dst, n), axis=1)

    return lax.cond(overflow, slow, fast, None)
